# Initial kernel scaffold; baseline (speedup 1.0000x reference)
import jax
import jax.numpy as jnp
from jax.experimental import pallas as pl


def kernel(x, embedding, affine_mean, affine_std):
    raise NotImplementedError("write your pallas kernel here")


# speedup vs baseline: 1.4153x; 1.4153x over previous
import functools

import jax
import jax.numpy as jnp
from jax.experimental import pallas as pl

N_ROWS = 32 * 1024
K_CODES = 1024
D = 64
BLOCK_R = 512


def _vq_block(x_ref, emb_ref, mean_ref, std_ref, q_ref, idx_ref, loss_ref):
    i = pl.program_id(0)
    x = x_ref[...]
    emb = mean_ref[...] + std_ref[...] * emb_ref[...]

    a2 = jnp.sum(x * x, axis=1, keepdims=True)
    b2 = jnp.sum(emb * emb, axis=1)
    xg = jax.lax.dot_general(x, emb, (((1,), (1,)), ((), ())),
                             preferred_element_type=jnp.float32)
    d2 = jnp.maximum(a2 + b2[None, :] - 2.0 * xg, 0.0)

    dmin = jnp.min(d2, axis=1, keepdims=True)
    lane = jax.lax.broadcasted_iota(jnp.int32, d2.shape, 1)
    idx = jnp.min(jnp.where(d2 <= dmin, lane, K_CODES), axis=1)

    onehot = (lane == idx[:, None]).astype(jnp.float32)
    q = jax.lax.dot_general(onehot, emb, (((1,), (0,)), ((), ())),
                            preferred_element_type=jnp.float32)

    idx_ref[...] = idx
    diff = q - x
    q_ref[...] = x + diff
    part = jnp.sum(diff * diff, axis=(0, 1), keepdims=True)

    @pl.when(i == 0)
    def _():
        loss_ref[...] = jnp.zeros_like(loss_ref)

    loss_ref[...] += part


@functools.partial(jax.jit, static_argnames=())
def kernel(x, embedding, affine_mean, affine_std):
    flat_x = x.reshape(-1, D)
    mean2 = affine_mean.reshape(1, D)
    std2 = affine_std.reshape(1, D)
    grid = (N_ROWS // BLOCK_R,)
    q, idx, loss_sum = pl.pallas_call(
        _vq_block,
        grid=grid,
        in_specs=[
            pl.BlockSpec((BLOCK_R, D), lambda i: (i, 0)),
            pl.BlockSpec((K_CODES, D), lambda i: (0, 0)),
            pl.BlockSpec((1, D), lambda i: (0, 0)),
            pl.BlockSpec((1, D), lambda i: (0, 0)),
        ],
        out_specs=[
            pl.BlockSpec((BLOCK_R, D), lambda i: (i, 0)),
            pl.BlockSpec((BLOCK_R,), lambda i: (i,)),
            pl.BlockSpec((1, 1), lambda i: (0, 0)),
        ],
        out_shape=[
            jax.ShapeDtypeStruct((N_ROWS, D), jnp.float32),
            jax.ShapeDtypeStruct((N_ROWS,), jnp.int32),
            jax.ShapeDtypeStruct((1, 1), jnp.float32),
        ],
    )(flat_x, embedding, mean2, std2)
    vq_loss = 2.0 * loss_sum[0, 0] / (N_ROWS * D)
    return q.reshape(x.shape), vq_loss, idx


# speedup vs baseline: 1.4389x; 1.0167x over previous
import functools

import jax
import jax.numpy as jnp
from jax.experimental import pallas as pl
from jax.experimental.pallas import tpu as pltpu

N_ROWS = 32 * 1024
K_CODES = 1024
D = 64
BLOCK_R = 512


def _vq_block(x_ref, emb_ref, mean_ref, std_ref, q_ref, idx_ref, loss_ref,
              emb_s, b2h_s):
    i = pl.program_id(0)

    @pl.when(i == 0)
    def _():
        emb = mean_ref[...] + std_ref[...] * emb_ref[...]
        emb_s[...] = emb
        b2h_s[...] = 0.5 * jnp.sum(emb * emb, axis=1)[None, :]

    x = x_ref[...]
    emb = emb_s[...]
    xg = jax.lax.dot_general(x, emb, (((1,), (1,)), ((), ())),
                             preferred_element_type=jnp.float32)
    score = b2h_s[...] - xg

    cmin = jnp.min(score, axis=1, keepdims=True)
    lane = jax.lax.broadcasted_iota(jnp.int32, score.shape, 1)
    idx = jnp.min(jnp.where(score <= cmin, lane, K_CODES), axis=1)

    onehot = (lane == idx[:, None]).astype(jnp.float32)
    q = jax.lax.dot_general(onehot, emb, (((1,), (0,)), ((), ())),
                            preferred_element_type=jnp.float32)

    idx_ref[...] = idx
    diff = q - x
    q_ref[...] = x + diff
    part = jnp.sum(diff * diff, axis=(0, 1), keepdims=True)

    @pl.when(i == 0)
    def _():
        loss_ref[...] = jnp.zeros_like(loss_ref)

    loss_ref[...] += part


@functools.partial(jax.jit, static_argnames=())
def kernel(x, embedding, affine_mean, affine_std):
    flat_x = x.reshape(-1, D)
    mean2 = affine_mean.reshape(1, D)
    std2 = affine_std.reshape(1, D)
    grid = (N_ROWS // BLOCK_R,)
    q, idx, loss_sum = pl.pallas_call(
        _vq_block,
        grid=grid,
        in_specs=[
            pl.BlockSpec((BLOCK_R, D), lambda i: (i, 0)),
            pl.BlockSpec((K_CODES, D), lambda i: (0, 0)),
            pl.BlockSpec((1, D), lambda i: (0, 0)),
            pl.BlockSpec((1, D), lambda i: (0, 0)),
        ],
        out_specs=[
            pl.BlockSpec((BLOCK_R, D), lambda i: (i, 0)),
            pl.BlockSpec((BLOCK_R,), lambda i: (i,)),
            pl.BlockSpec((1, 1), lambda i: (0, 0)),
        ],
        out_shape=[
            jax.ShapeDtypeStruct((N_ROWS, D), jnp.float32),
            jax.ShapeDtypeStruct((N_ROWS,), jnp.int32),
            jax.ShapeDtypeStruct((1, 1), jnp.float32),
        ],
        scratch_shapes=[
            pltpu.VMEM((K_CODES, D), jnp.float32),
            pltpu.VMEM((1, K_CODES), jnp.float32),
        ],
    )(flat_x, embedding, mean2, std2)
    vq_loss = 2.0 * loss_sum[0, 0] / (N_ROWS * D)
    return q.reshape(x.shape), vq_loss, idx


# speedup vs baseline: 1.5947x; 1.1083x over previous
import functools

import jax
import jax.numpy as jnp
from jax.experimental import pallas as pl
from jax.experimental.pallas import tpu as pltpu

N_ROWS = 32 * 1024
K_CODES = 1024
D = 64
BLOCK_R = 1024


def _vq_block(x_ref, emb_ref, mean_ref, std_ref, q_ref, idx_ref, loss_ref,
              emb_s, b2h_s):
    i = pl.program_id(0)

    @pl.when(i == 0)
    def _():
        emb = mean_ref[...] + std_ref[...] * emb_ref[...]
        emb_s[...] = emb
        b2h_s[...] = 0.5 * jnp.sum(emb * emb, axis=1)[None, :]

    x = x_ref[...]
    emb = emb_s[...]
    xg = jax.lax.dot_general(x, emb, (((1,), (1,)), ((), ())),
                             preferred_element_type=jnp.float32)
    score = b2h_s[...] - xg

    cmin = jnp.min(score, axis=1, keepdims=True)
    lane = jax.lax.broadcasted_iota(jnp.int32, score.shape, 1)
    idx = jnp.min(jnp.where(score <= cmin, lane, K_CODES), axis=1)

    onehot = (lane == idx[:, None]).astype(jnp.float32)
    q = jax.lax.dot_general(onehot, emb, (((1,), (0,)), ((), ())),
                            preferred_element_type=jnp.float32)

    idx_ref[...] = idx
    diff = q - x
    q_ref[...] = x + diff
    part = jnp.sum(diff * diff, axis=(0, 1), keepdims=True)

    @pl.when(i == 0)
    def _():
        loss_ref[...] = jnp.zeros_like(loss_ref)

    loss_ref[...] += part


@functools.partial(jax.jit, static_argnames=())
def kernel(x, embedding, affine_mean, affine_std):
    flat_x = x.reshape(-1, D)
    mean2 = affine_mean.reshape(1, D)
    std2 = affine_std.reshape(1, D)
    grid = (N_ROWS // BLOCK_R,)
    q, idx, loss_sum = pl.pallas_call(
        _vq_block,
        grid=grid,
        in_specs=[
            pl.BlockSpec((BLOCK_R, D), lambda i: (i, 0)),
            pl.BlockSpec((K_CODES, D), lambda i: (0, 0)),
            pl.BlockSpec((1, D), lambda i: (0, 0)),
            pl.BlockSpec((1, D), lambda i: (0, 0)),
        ],
        out_specs=[
            pl.BlockSpec((BLOCK_R, D), lambda i: (i, 0)),
            pl.BlockSpec((BLOCK_R,), lambda i: (i,)),
            pl.BlockSpec((1, 1), lambda i: (0, 0)),
        ],
        out_shape=[
            jax.ShapeDtypeStruct((N_ROWS, D), jnp.float32),
            jax.ShapeDtypeStruct((N_ROWS,), jnp.int32),
            jax.ShapeDtypeStruct((1, 1), jnp.float32),
        ],
        scratch_shapes=[
            pltpu.VMEM((K_CODES, D), jnp.float32),
            pltpu.VMEM((1, K_CODES), jnp.float32),
        ],
    )(flat_x, embedding, mean2, std2)
    vq_loss = 2.0 * loss_sum[0, 0] / (N_ROWS * D)
    return q.reshape(x.shape), vq_loss, idx


# speedup vs baseline: 1.6779x; 1.0522x over previous
import functools

import jax
import jax.numpy as jnp
from jax.experimental import pallas as pl
from jax.experimental.pallas import tpu as pltpu

N_ROWS = 32 * 1024
K_CODES = 1024
D = 64
BLOCK_R = 2048


def _vq_block(x_ref, emb_ref, mean_ref, std_ref, q_ref, idx_ref, loss_ref,
              emb_s, b2h_s):
    i = pl.program_id(0)

    @pl.when(i == 0)
    def _():
        emb = mean_ref[...] + std_ref[...] * emb_ref[...]
        emb_s[...] = emb
        b2h_s[...] = 0.5 * jnp.sum(emb * emb, axis=1)[None, :]

    x = x_ref[...]
    emb = emb_s[...]
    xg = jax.lax.dot_general(x, emb, (((1,), (1,)), ((), ())),
                             preferred_element_type=jnp.float32)
    score = b2h_s[...] - xg

    cmin = jnp.min(score, axis=1, keepdims=True)
    lane = jax.lax.broadcasted_iota(jnp.int32, score.shape, 1)
    idx = jnp.min(jnp.where(score <= cmin, lane, K_CODES), axis=1)

    onehot = (lane == idx[:, None]).astype(jnp.float32)
    q = jax.lax.dot_general(onehot, emb, (((1,), (0,)), ((), ())),
                            preferred_element_type=jnp.float32)

    idx_ref[...] = idx
    diff = q - x
    q_ref[...] = x + diff
    part = jnp.sum(diff * diff, axis=(0, 1), keepdims=True)

    @pl.when(i == 0)
    def _():
        loss_ref[...] = jnp.zeros_like(loss_ref)

    loss_ref[...] += part


@functools.partial(jax.jit, static_argnames=())
def kernel(x, embedding, affine_mean, affine_std):
    flat_x = x.reshape(-1, D)
    mean2 = affine_mean.reshape(1, D)
    std2 = affine_std.reshape(1, D)
    grid = (N_ROWS // BLOCK_R,)
    q, idx, loss_sum = pl.pallas_call(
        _vq_block,
        grid=grid,
        in_specs=[
            pl.BlockSpec((BLOCK_R, D), lambda i: (i, 0)),
            pl.BlockSpec((K_CODES, D), lambda i: (0, 0)),
            pl.BlockSpec((1, D), lambda i: (0, 0)),
            pl.BlockSpec((1, D), lambda i: (0, 0)),
        ],
        out_specs=[
            pl.BlockSpec((BLOCK_R, D), lambda i: (i, 0)),
            pl.BlockSpec((BLOCK_R,), lambda i: (i,)),
            pl.BlockSpec((1, 1), lambda i: (0, 0)),
        ],
        out_shape=[
            jax.ShapeDtypeStruct((N_ROWS, D), jnp.float32),
            jax.ShapeDtypeStruct((N_ROWS,), jnp.int32),
            jax.ShapeDtypeStruct((1, 1), jnp.float32),
        ],
        scratch_shapes=[
            pltpu.VMEM((K_CODES, D), jnp.float32),
            pltpu.VMEM((1, K_CODES), jnp.float32),
        ],
    )(flat_x, embedding, mean2, std2)
    vq_loss = 2.0 * loss_sum[0, 0] / (N_ROWS * D)
    return q.reshape(x.shape), vq_loss, idx


# speedup vs baseline: 1.7162x; 1.0229x over previous
import functools

import jax
import jax.numpy as jnp
from jax.experimental import pallas as pl
from jax.experimental.pallas import tpu as pltpu

N_ROWS = 32 * 1024
K_CODES = 1024
D = 64
BLOCK_R = 4096


def _vq_block(x_ref, emb_ref, mean_ref, std_ref, q_ref, idx_ref, loss_ref,
              emb_s, b2h_s):
    i = pl.program_id(0)

    @pl.when(i == 0)
    def _():
        emb = mean_ref[...] + std_ref[...] * emb_ref[...]
        emb_s[...] = emb
        b2h_s[...] = 0.5 * jnp.sum(emb * emb, axis=1)[None, :]

    x = x_ref[...]
    emb = emb_s[...]
    xg = jax.lax.dot_general(x, emb, (((1,), (1,)), ((), ())),
                             preferred_element_type=jnp.float32)
    score = b2h_s[...] - xg

    cmin = jnp.min(score, axis=1, keepdims=True)
    lane = jax.lax.broadcasted_iota(jnp.int32, score.shape, 1)
    idx = jnp.min(jnp.where(score <= cmin, lane, K_CODES), axis=1)

    onehot = (lane == idx[:, None]).astype(jnp.float32)
    q = jax.lax.dot_general(onehot, emb, (((1,), (0,)), ((), ())),
                            preferred_element_type=jnp.float32)

    idx_ref[...] = idx
    diff = q - x
    q_ref[...] = x + diff
    part = jnp.sum(diff * diff, axis=(0, 1), keepdims=True)

    @pl.when(i == 0)
    def _():
        loss_ref[...] = jnp.zeros_like(loss_ref)

    loss_ref[...] += part


@functools.partial(jax.jit, static_argnames=())
def kernel(x, embedding, affine_mean, affine_std):
    flat_x = x.reshape(-1, D)
    mean2 = affine_mean.reshape(1, D)
    std2 = affine_std.reshape(1, D)
    grid = (N_ROWS // BLOCK_R,)
    q, idx, loss_sum = pl.pallas_call(
        _vq_block,
        grid=grid,
        in_specs=[
            pl.BlockSpec((BLOCK_R, D), lambda i: (i, 0)),
            pl.BlockSpec((K_CODES, D), lambda i: (0, 0)),
            pl.BlockSpec((1, D), lambda i: (0, 0)),
            pl.BlockSpec((1, D), lambda i: (0, 0)),
        ],
        out_specs=[
            pl.BlockSpec((BLOCK_R, D), lambda i: (i, 0)),
            pl.BlockSpec((BLOCK_R,), lambda i: (i,)),
            pl.BlockSpec((1, 1), lambda i: (0, 0)),
        ],
        out_shape=[
            jax.ShapeDtypeStruct((N_ROWS, D), jnp.float32),
            jax.ShapeDtypeStruct((N_ROWS,), jnp.int32),
            jax.ShapeDtypeStruct((1, 1), jnp.float32),
        ],
        scratch_shapes=[
            pltpu.VMEM((K_CODES, D), jnp.float32),
            pltpu.VMEM((1, K_CODES), jnp.float32),
        ],
    )(flat_x, embedding, mean2, std2)
    vq_loss = 2.0 * loss_sum[0, 0] / (N_ROWS * D)
    return q.reshape(x.shape), vq_loss, idx


# speedup vs baseline: 1.7249x; 1.0050x over previous
import functools

import jax
import jax.numpy as jnp
from jax.experimental import pallas as pl
from jax.experimental.pallas import tpu as pltpu

N_ROWS = 32 * 1024
K_CODES = 1024
D = 64
BLOCK_R = 8192


def _vq_block(x_ref, emb_ref, mean_ref, std_ref, q_ref, idx_ref, loss_ref,
              emb_s, b2h_s):
    i = pl.program_id(0)

    @pl.when(i == 0)
    def _():
        emb = mean_ref[...] + std_ref[...] * emb_ref[...]
        emb_s[...] = emb
        b2h_s[...] = 0.5 * jnp.sum(emb * emb, axis=1)[None, :]

    x = x_ref[...]
    emb = emb_s[...]
    xg = jax.lax.dot_general(x, emb, (((1,), (1,)), ((), ())),
                             preferred_element_type=jnp.float32)
    score = b2h_s[...] - xg

    cmin = jnp.min(score, axis=1, keepdims=True)
    lane = jax.lax.broadcasted_iota(jnp.int32, score.shape, 1)
    idx = jnp.min(jnp.where(score <= cmin, lane, K_CODES), axis=1)

    onehot = (lane == idx[:, None]).astype(jnp.float32)
    q = jax.lax.dot_general(onehot, emb, (((1,), (0,)), ((), ())),
                            preferred_element_type=jnp.float32)

    idx_ref[...] = idx
    diff = q - x
    q_ref[...] = x + diff
    part = jnp.sum(diff * diff, axis=(0, 1), keepdims=True)

    @pl.when(i == 0)
    def _():
        loss_ref[...] = jnp.zeros_like(loss_ref)

    loss_ref[...] += part


@functools.partial(jax.jit, static_argnames=())
def kernel(x, embedding, affine_mean, affine_std):
    flat_x = x.reshape(-1, D)
    mean2 = affine_mean.reshape(1, D)
    std2 = affine_std.reshape(1, D)
    grid = (N_ROWS // BLOCK_R,)
    q, idx, loss_sum = pl.pallas_call(
        _vq_block,
        grid=grid,
        in_specs=[
            pl.BlockSpec((BLOCK_R, D), lambda i: (i, 0)),
            pl.BlockSpec((K_CODES, D), lambda i: (0, 0)),
            pl.BlockSpec((1, D), lambda i: (0, 0)),
            pl.BlockSpec((1, D), lambda i: (0, 0)),
        ],
        out_specs=[
            pl.BlockSpec((BLOCK_R, D), lambda i: (i, 0)),
            pl.BlockSpec((BLOCK_R,), lambda i: (i,)),
            pl.BlockSpec((1, 1), lambda i: (0, 0)),
        ],
        out_shape=[
            jax.ShapeDtypeStruct((N_ROWS, D), jnp.float32),
            jax.ShapeDtypeStruct((N_ROWS,), jnp.int32),
            jax.ShapeDtypeStruct((1, 1), jnp.float32),
        ],
        scratch_shapes=[
            pltpu.VMEM((K_CODES, D), jnp.float32),
            pltpu.VMEM((1, K_CODES), jnp.float32),
        ],
    )(flat_x, embedding, mean2, std2)
    vq_loss = 2.0 * loss_sum[0, 0] / (N_ROWS * D)
    return q.reshape(x.shape), vq_loss, idx
